# baseline (speedup 1.0000x reference)
import functools

import jax
import jax.numpy as jnp
import numpy as np
from jax import lax
from jax.experimental import pallas as pl
from jax.experimental.pallas import tpu as pltpu
from jax.experimental.pallas import tpu_sc as plsc

D = 16
NC = 2
NS = 16
NW = NC * NS
LANE_TILES = 4
FP = LANE_TILES * 8


def _make_gather(n_slots, c):
    per_w = n_slots // NW
    n_chunks = per_w // c
    mesh = plsc.VectorSubcoreMesh(core_axis_name="c", subcore_axis_name="s")

    @functools.partial(
        pl.kernel,
        mesh=mesh,
        compiler_params=pltpu.CompilerParams(use_tc_tiling_on_sc=False),
        out_type=[
            jax.ShapeDtypeStruct((n_slots, D), jnp.float32),
            jax.ShapeDtypeStruct((n_slots, D), jnp.float32),
        ],
        scratch_types=[
            pltpu.VMEM((c,), jnp.int32),
            pltpu.VMEM((c,), jnp.int32),
            pltpu.VMEM((c, D), jnp.float32),
            pltpu.VMEM((c, D), jnp.float32),
            pltpu.VMEM((c, D), jnp.float32),
            pltpu.VMEM((c, D), jnp.float32),
            pltpu.SemaphoreType.DMA,
            pltpu.SemaphoreType.DMA,
            pltpu.SemaphoreType.DMA,
            pltpu.SemaphoreType.DMA,
        ],
    )
    def gather_k(idx_hbm, deep_hbm, wide_hbm, deep_out, wide_out,
                 i0, i1, d0, d1, w0, w1, sd0, sd1, sw0, sw1):
        wid = lax.axis_index("s") * NC + lax.axis_index("c")
        base = wid * per_w
        r_per_chunk = c // 1024
        ibufs = (i0, i1)
        dbufs = (d0, d1)
        wbufs = (w0, w1)
        sds = (sd0, sd1)
        sws = (sw0, sw1)
        handles = [None, None]

        def fire(g):
            b = g & 1
            off = base + g * c
            pltpu.sync_copy(idx_hbm.at[pl.ds(off, c)], ibufs[b])
            hd = pltpu.async_copy(deep_hbm.at[ibufs[b]], dbufs[b], sds[b])
            hw = pltpu.async_copy(wide_hbm.at[ibufs[b]], wbufs[b], sws[b])
            handles[b] = (hd, hw)

        fire(0)
        for g in range(n_chunks):
            if g + 1 < n_chunks:
                fire(g + 1)
            hd, hw = handles[g & 1]
            hd.wait()
            hw.wait()
            off = base + g * c
            pltpu.sync_copy(dbufs[g & 1], deep_out.at[pl.ds(off, c)])
            pltpu.sync_copy(wbufs[g & 1], wide_out.at[pl.ds(off, c)])

    return gather_k


def _softplus(x):
    return jnp.maximum(x, 0.0) + jnp.log1p(jnp.exp(-jnp.abs(x)))


def _dense_body(de, we, g_r, bt_r, w1_r, b1_r, w2_r, b2_r, wh1_r, bh1_r,
                wh2_r, bh2_r, ww_r, prop_o, k_o, l_o, wide_o):
    bbr = de.shape[0]
    bb = bbr * 8
    inn = 416.0
    ej = [de[:, j].reshape(bb, 128) for j in range(LANE_TILES)]
    wj = [we[:, j].reshape(bb, 128) for j in range(LANE_TILES)]
    lane = lax.broadcasted_iota(jnp.int32, (bb, 128), 1)
    m3 = (lane < 32).astype(jnp.float32)

    s = (jnp.sum(ej[0], axis=-1, keepdims=True)
         + jnp.sum(ej[1], axis=-1, keepdims=True)
         + jnp.sum(ej[2], axis=-1, keepdims=True)
         + jnp.sum(ej[3] * m3, axis=-1, keepdims=True))
    mu = s / inn
    v = (jnp.sum((ej[0] - mu) ** 2, axis=-1, keepdims=True)
         + jnp.sum((ej[1] - mu) ** 2, axis=-1, keepdims=True)
         + jnp.sum((ej[2] - mu) ** 2, axis=-1, keepdims=True)
         + jnp.sum(((ej[3] - mu) ** 2) * m3, axis=-1, keepdims=True))
    var = v / inn
    rstd = 1.0 / jnp.sqrt(var + 1e-5)

    h = b1_r[...]
    wide = jnp.zeros((bb, 1), jnp.float32)
    for j in range(LANE_TILES):
        hj = (ej[j] - mu) * rstd * g_r[j] + bt_r[j]
        h = h + jnp.dot(hj, w1_r[j], preferred_element_type=jnp.float32)
        wide = wide + jnp.sum(wj[j] * ww_r[j], axis=-1, keepdims=True)
    h = jnp.maximum(h, 0.0)
    h = jnp.maximum(jnp.dot(h, w2_r[...], preferred_element_type=jnp.float32)
                    + b2_r[...], 0.0)
    h3 = jnp.maximum(jnp.dot(h, wh1_r[...], preferred_element_type=jnp.float32)
                     + bh1_r[...], 0.0)
    out3 = jnp.dot(h3, wh2_r[...], preferred_element_type=jnp.float32) + bh2_r[...]
    prop_o[...] = jax.nn.sigmoid(out3[:, 0:1])
    k_o[...] = jnp.maximum(_softplus(out3[:, 1:2]), 0.01)
    l_o[...] = jnp.maximum(_softplus(out3[:, 2:3]), 0.01)
    wide_o[...] = wide


def _dense_call(deep4, wide4, g4, bt4, W1p, b1, W2, b2,
                Wh1, bh1, Wh2, bh2, Wwp, bb):
    nr = deep4.shape[0]
    b = nr * 8
    bbr = bb // 8
    grid = (b // bb,)

    def full(shape):
        return pl.BlockSpec(shape, lambda i: tuple(0 for _ in shape))

    return pl.pallas_call(
        _dense_body,
        grid=grid,
        in_specs=[
            pl.BlockSpec((bbr, LANE_TILES, 8, 128), lambda i: (i, 0, 0, 0)),
            pl.BlockSpec((bbr, LANE_TILES, 8, 128), lambda i: (i, 0, 0, 0)),
            full((LANE_TILES, 128)), full((LANE_TILES, 128)),
            full((LANE_TILES, 128, 512)), full((512,)),
            full((512, 256)), full((256,)),
            full((256, 384)), full((384,)),
            full((384, 3)), full((3,)),
            full((LANE_TILES, 128)),
        ],
        out_specs=[
            pl.BlockSpec((bb, 1), lambda i: (i, 0)),
            pl.BlockSpec((bb, 1), lambda i: (i, 0)),
            pl.BlockSpec((bb, 1), lambda i: (i, 0)),
            pl.BlockSpec((bb, 1), lambda i: (i, 0)),
        ],
        out_shape=[jax.ShapeDtypeStruct((b, 1), jnp.float32)] * 4,
    )(deep4, wide4, g4, bt4, W1p, b1, W2, b2, Wh1, bh1, Wh2, bh2, Wwp)


def kernel(x, deep_table, wide_table, ln_gamma, ln_beta, W1, b1, W2, b2,
           Wp1, bp1, Wp2, bp2, Wk1, bk1, Wk2, bk2, Wl1, bl1, Wl2, bl2, Wwide):
    b, f = x.shape
    inn = f * D
    padl = LANE_TILES * 128 - inn
    xpad = jnp.pad(x, ((0, 0), (0, FP - f)))
    pidx = (xpad.reshape(b // 8, 8, LANE_TILES, 8)
            .transpose(0, 2, 1, 3).reshape(-1))
    n_slots = b * FP

    gather = _make_gather(n_slots, 1024)
    deep2, wide2 = gather(pidx, deep_table, wide_table)
    deep4 = deep2.reshape(b // 8, LANE_TILES, 8, 128)
    wide4 = wide2.reshape(b // 8, LANE_TILES, 8, 128)

    W1p = jnp.pad(W1, ((0, padl), (0, 0))).reshape(LANE_TILES, 128, 512)
    Wwp = jnp.pad(Wwide[:, 0], (0, padl)).reshape(LANE_TILES, 128)
    g4 = jnp.pad(ln_gamma, (0, padl)).reshape(LANE_TILES, 128)
    bt4 = jnp.pad(ln_beta, (0, padl)).reshape(LANE_TILES, 128)

    Wh1 = jnp.concatenate([Wp1, Wk1, Wl1], axis=1)
    bh1 = jnp.concatenate([bp1, bk1, bl1], axis=0)
    z = jnp.zeros((128, 1), jnp.float32)
    Wh2 = jnp.concatenate([
        jnp.concatenate([Wp2, z, z], axis=1),
        jnp.concatenate([z, Wk2, z], axis=1),
        jnp.concatenate([z, z, Wl2], axis=1),
    ], axis=0)
    bh2 = jnp.concatenate([bp2, bk2, bl2], axis=0)

    prop, k_p, l_p, wide = _dense_call(
        deep4, wide4, g4, bt4, W1p, b1, W2, b2,
        Wh1, bh1, Wh2, bh2, Wwp, 512)
    return (prop, k_p, l_p, wide)


# speedup vs baseline: 1.4835x; 1.4835x over previous
import functools

import jax
import jax.numpy as jnp
import numpy as np
from jax import lax
from jax.experimental import pallas as pl
from jax.experimental.pallas import tpu as pltpu
from jax.experimental.pallas import tpu_sc as plsc

D = 16
NC = 2
NS = 16
NW = NC * NS
LANE_TILES = 4
FP = LANE_TILES * 8


def _make_gather(n_slots, c):
    per_w = n_slots // NW
    n_chunks = per_w // c
    mesh = plsc.VectorSubcoreMesh(core_axis_name="c", subcore_axis_name="s")

    @functools.partial(
        pl.kernel,
        mesh=mesh,
        compiler_params=pltpu.CompilerParams(use_tc_tiling_on_sc=False),
        out_type=[
            jax.ShapeDtypeStruct((n_slots, D), jnp.float32),
            jax.ShapeDtypeStruct((n_slots, D), jnp.float32),
        ],
        scratch_types=[
            pltpu.VMEM((c,), jnp.int32),
            pltpu.VMEM((c,), jnp.int32),
            pltpu.VMEM((c, D), jnp.float32),
            pltpu.VMEM((c, D), jnp.float32),
            pltpu.VMEM((c, D), jnp.float32),
            pltpu.VMEM((c, D), jnp.float32),
            pltpu.SemaphoreType.DMA,
            pltpu.SemaphoreType.DMA,
            pltpu.SemaphoreType.DMA,
            pltpu.SemaphoreType.DMA,
        ],
    )
    def gather_k(idx_hbm, deep_hbm, wide_hbm, deep_out, wide_out,
                 i0, i1, d0, d1, w0, w1, sd0, sd1, sw0, sw1):
        wid = lax.axis_index("s") * NC + lax.axis_index("c")
        base = wid * per_w
        r_per_chunk = c // 1024
        ibufs = (i0, i1)
        dbufs = (d0, d1)
        wbufs = (w0, w1)
        sds = (sd0, sd1)
        sws = (sw0, sw1)
        handles = [None, None]

        def fire(g):
            b = g & 1
            off = base + g * c
            pltpu.sync_copy(idx_hbm.at[pl.ds(off, c)], ibufs[b])
            hd = pltpu.async_copy(deep_hbm.at[ibufs[b]], dbufs[b], sds[b])
            hw = pltpu.async_copy(wide_hbm.at[ibufs[b]], wbufs[b], sws[b])
            handles[b] = (hd, hw)

        fire(0)
        for g in range(n_chunks):
            if g + 1 < n_chunks:
                fire(g + 1)
            hd, hw = handles[g & 1]
            hd.wait()
            hw.wait()
            off = base + g * c
            pltpu.sync_copy(dbufs[g & 1], deep_out.at[pl.ds(off, c)])
            pltpu.sync_copy(wbufs[g & 1], wide_out.at[pl.ds(off, c)])

    return gather_k


def _softplus(x):
    return jnp.maximum(x, 0.0) + jnp.log1p(jnp.exp(-jnp.abs(x)))


def _dense_body(de, we, g_r, bt_r, w1_r, b1_r, w2_r, b2_r, wh1_r, bh1_r,
                wh2_r, bh2_r, ww_r, prop_o, k_o, l_o, wide_o):
    bbr = de.shape[0]
    bb = bbr * 8
    inn = 416.0
    ej = [de[:, j].reshape(bb, 128) for j in range(LANE_TILES)]
    wj = [we[:, j].reshape(bb, 128) for j in range(LANE_TILES)]
    lane = lax.broadcasted_iota(jnp.int32, (bb, 128), 1)
    m3 = (lane < 32).astype(jnp.float32)

    s = (jnp.sum(ej[0], axis=-1, keepdims=True)
         + jnp.sum(ej[1], axis=-1, keepdims=True)
         + jnp.sum(ej[2], axis=-1, keepdims=True)
         + jnp.sum(ej[3] * m3, axis=-1, keepdims=True))
    mu = s / inn
    v = (jnp.sum((ej[0] - mu) ** 2, axis=-1, keepdims=True)
         + jnp.sum((ej[1] - mu) ** 2, axis=-1, keepdims=True)
         + jnp.sum((ej[2] - mu) ** 2, axis=-1, keepdims=True)
         + jnp.sum(((ej[3] - mu) ** 2) * m3, axis=-1, keepdims=True))
    var = v / inn
    rstd = 1.0 / jnp.sqrt(var + 1e-5)

    h = b1_r[...]
    wide = jnp.zeros((bb, 1), jnp.float32)
    for j in range(LANE_TILES):
        hj = (ej[j] - mu) * rstd * g_r[j] + bt_r[j]
        h = h + jnp.dot(hj, w1_r[j], preferred_element_type=jnp.float32)
        wide = wide + jnp.sum(wj[j] * ww_r[j], axis=-1, keepdims=True)
    h = jnp.maximum(h, 0.0)
    h = jnp.maximum(jnp.dot(h, w2_r[...], preferred_element_type=jnp.float32)
                    + b2_r[...], 0.0)
    h3 = jnp.maximum(jnp.dot(h, wh1_r[...], preferred_element_type=jnp.float32)
                     + bh1_r[...], 0.0)
    out3 = jnp.dot(h3, wh2_r[...], preferred_element_type=jnp.float32) + bh2_r[...]
    prop_o[...] = jax.nn.sigmoid(out3[:, 0:1])
    k_o[...] = jnp.maximum(_softplus(out3[:, 1:2]), 0.01)
    l_o[...] = jnp.maximum(_softplus(out3[:, 2:3]), 0.01)
    wide_o[...] = wide


def _dense_call(deep4, wide4, g4, bt4, W1p, b1, W2, b2,
                Wh1, bh1, Wh2, bh2, Wwp, bb):
    nr = deep4.shape[0]
    b = nr * 8
    bbr = bb // 8
    grid = (b // bb,)

    def full(shape):
        return pl.BlockSpec(shape, lambda i: tuple(0 for _ in shape))

    return pl.pallas_call(
        _dense_body,
        grid=grid,
        in_specs=[
            pl.BlockSpec((bbr, LANE_TILES, 8, 128), lambda i: (i, 0, 0, 0)),
            pl.BlockSpec((bbr, LANE_TILES, 8, 128), lambda i: (i, 0, 0, 0)),
            full((LANE_TILES, 128)), full((LANE_TILES, 128)),
            full((LANE_TILES, 128, 512)), full((512,)),
            full((512, 256)), full((256,)),
            full((256, 384)), full((384,)),
            full((384, 3)), full((3,)),
            full((LANE_TILES, 128)),
        ],
        out_specs=[
            pl.BlockSpec((bb, 1), lambda i: (i, 0)),
            pl.BlockSpec((bb, 1), lambda i: (i, 0)),
            pl.BlockSpec((bb, 1), lambda i: (i, 0)),
            pl.BlockSpec((bb, 1), lambda i: (i, 0)),
        ],
        out_shape=[jax.ShapeDtypeStruct((b, 1), jnp.float32)] * 4,
    )(deep4, wide4, g4, bt4, W1p, b1, W2, b2, Wh1, bh1, Wh2, bh2, Wwp)


def kernel(x, deep_table, wide_table, ln_gamma, ln_beta, W1, b1, W2, b2,
           Wp1, bp1, Wp2, bp2, Wk1, bk1, Wk2, bk2, Wl1, bl1, Wl2, bl2, Wwide):
    b, f = x.shape
    inn = f * D
    padl = LANE_TILES * 128 - inn
    xpad = jnp.concatenate([x, x[:, : FP - f]], axis=1)
    pidx = (xpad.reshape(b // 8, 8, LANE_TILES, 8)
            .transpose(0, 2, 1, 3).reshape(-1))
    n_slots = b * FP

    gather = _make_gather(n_slots, 1024)
    deep2, wide2 = gather(pidx, deep_table, wide_table)
    deep4 = deep2.reshape(b // 8, LANE_TILES, 8, 128)
    wide4 = wide2.reshape(b // 8, LANE_TILES, 8, 128)

    W1p = jnp.pad(W1, ((0, padl), (0, 0))).reshape(LANE_TILES, 128, 512)
    Wwp = jnp.pad(Wwide[:, 0], (0, padl)).reshape(LANE_TILES, 128)
    g4 = jnp.pad(ln_gamma, (0, padl)).reshape(LANE_TILES, 128)
    bt4 = jnp.pad(ln_beta, (0, padl)).reshape(LANE_TILES, 128)

    Wh1 = jnp.concatenate([Wp1, Wk1, Wl1], axis=1)
    bh1 = jnp.concatenate([bp1, bk1, bl1], axis=0)
    z = jnp.zeros((128, 1), jnp.float32)
    Wh2 = jnp.concatenate([
        jnp.concatenate([Wp2, z, z], axis=1),
        jnp.concatenate([z, Wk2, z], axis=1),
        jnp.concatenate([z, z, Wl2], axis=1),
    ], axis=0)
    bh2 = jnp.concatenate([bp2, bk2, bl2], axis=0)

    prop, k_p, l_p, wide = _dense_call(
        deep4, wide4, g4, bt4, W1p, b1, W2, b2,
        Wh1, bh1, Wh2, bh2, Wwp, 512)
    return (prop, k_p, l_p, wide)


# speedup vs baseline: 2.0245x; 1.3646x over previous
import functools

import jax
import jax.numpy as jnp
import numpy as np
from jax import lax
from jax.experimental import pallas as pl
from jax.experimental.pallas import tpu as pltpu
from jax.experimental.pallas import tpu_sc as plsc

D = 16
NC = 2
NS = 16
NW = NC * NS
LANE_TILES = 4
FP = LANE_TILES * 8


def _make_gather(n_slots, c):
    per_w = n_slots // NW
    n_chunks = per_w // c
    mesh = plsc.VectorSubcoreMesh(core_axis_name="c", subcore_axis_name="s")

    @functools.partial(
        pl.kernel,
        mesh=mesh,
        compiler_params=pltpu.CompilerParams(use_tc_tiling_on_sc=False),
        out_type=[
            jax.ShapeDtypeStruct((n_slots, D), jnp.float32),
            jax.ShapeDtypeStruct((n_slots, D), jnp.float32),
        ],
        scratch_types=[
            pltpu.VMEM((c,), jnp.int32),
            pltpu.VMEM((c,), jnp.int32),
            pltpu.VMEM((c, D), jnp.float32),
            pltpu.VMEM((c, D), jnp.float32),
            pltpu.VMEM((c, D), jnp.float32),
            pltpu.VMEM((c, D), jnp.float32),
            pltpu.SemaphoreType.DMA,
            pltpu.SemaphoreType.DMA,
            pltpu.SemaphoreType.DMA,
            pltpu.SemaphoreType.DMA,
        ],
    )
    def gather_k(idx_hbm, deep_hbm, wide_hbm, deep_out, wide_out,
                 i0, i1, d0, d1, w0, w1, sd0, sd1, sw0, sw1):
        wid = lax.axis_index("s") * NC + lax.axis_index("c")
        base = wid * per_w
        r_per_chunk = c // 1024
        ibufs = (i0, i1)
        dbufs = (d0, d1)
        wbufs = (w0, w1)
        sds = (sd0, sd1)
        sws = (sw0, sw1)
        handles = [None, None]

        def fire(g):
            b = g & 1
            off = base + g * c
            pltpu.sync_copy(idx_hbm.at[pl.ds(off, c)], ibufs[b])
            hd = pltpu.async_copy(deep_hbm.at[ibufs[b]], dbufs[b], sds[b])
            hw = pltpu.async_copy(wide_hbm.at[ibufs[b]], wbufs[b], sws[b])
            handles[b] = (hd, hw)

        fire(0)
        for g in range(n_chunks):
            if g + 1 < n_chunks:
                fire(g + 1)
            hd, hw = handles[g & 1]
            hd.wait()
            hw.wait()
            off = base + g * c
            pltpu.sync_copy(dbufs[g & 1], deep_out.at[pl.ds(off, c)])
            pltpu.sync_copy(wbufs[g & 1], wide_out.at[pl.ds(off, c)])

    return gather_k


RL_LANES = 4096


def _relayout_body(t_r, e_r, o_r):
    x = t_r[...]
    acc = jnp.zeros(o_r.shape, jnp.float32)
    for u in range(8):
        xu = x[:, u * 512:(u + 1) * 512]
        acc = acc + lax.dot_general(
            xu, e_r[u], (((0,), (0,)), ((), ())),
            preferred_element_type=jnp.float32)
    o_r[...] = acc


def _relayout(table_t, eye8):
    d, v = table_t.shape
    grid = ((v + RL_LANES - 1) // RL_LANES,)
    rows = RL_LANES * d // 128
    return pl.pallas_call(
        _relayout_body,
        grid=grid,
        in_specs=[
            pl.BlockSpec((d, RL_LANES), lambda g: (0, g)),
            pl.BlockSpec((8, d, 128), lambda g: (0, 0, 0)),
        ],
        out_specs=pl.BlockSpec((rows, 128), lambda g: (g, 0)),
        out_shape=jax.ShapeDtypeStruct((grid[0] * rows, 128), jnp.float32),
    )(table_t, eye8)


def _swizzle(r):
    rl = r & (RL_LANES - 1)
    return (r & ~(RL_LANES - 1)) | ((rl & 511) << 3) | (rl >> 9)


def _softplus(x):
    return jnp.maximum(x, 0.0) + jnp.log1p(jnp.exp(-jnp.abs(x)))


def _dense_body(de, we, g_r, bt_r, w1_r, b1_r, w2_r, b2_r, wh1_r, bh1_r,
                wh2_r, bh2_r, ww_r, prop_o, k_o, l_o, wide_o):
    bbr = de.shape[0]
    bb = bbr * 8
    inn = 416.0
    ej = [de[:, j].reshape(bb, 128) for j in range(LANE_TILES)]
    wj = [we[:, j].reshape(bb, 128) for j in range(LANE_TILES)]
    lane = lax.broadcasted_iota(jnp.int32, (bb, 128), 1)
    m3 = (lane < 32).astype(jnp.float32)

    s = (jnp.sum(ej[0], axis=-1, keepdims=True)
         + jnp.sum(ej[1], axis=-1, keepdims=True)
         + jnp.sum(ej[2], axis=-1, keepdims=True)
         + jnp.sum(ej[3] * m3, axis=-1, keepdims=True))
    mu = s / inn
    v = (jnp.sum((ej[0] - mu) ** 2, axis=-1, keepdims=True)
         + jnp.sum((ej[1] - mu) ** 2, axis=-1, keepdims=True)
         + jnp.sum((ej[2] - mu) ** 2, axis=-1, keepdims=True)
         + jnp.sum(((ej[3] - mu) ** 2) * m3, axis=-1, keepdims=True))
    var = v / inn
    rstd = 1.0 / jnp.sqrt(var + 1e-5)

    h = b1_r[...]
    wide = jnp.zeros((bb, 1), jnp.float32)
    for j in range(LANE_TILES):
        hj = (ej[j] - mu) * rstd * g_r[j] + bt_r[j]
        h = h + jnp.dot(hj, w1_r[j], preferred_element_type=jnp.float32)
        wide = wide + jnp.sum(wj[j] * ww_r[j], axis=-1, keepdims=True)
    h = jnp.maximum(h, 0.0)
    h = jnp.maximum(jnp.dot(h, w2_r[...], preferred_element_type=jnp.float32)
                    + b2_r[...], 0.0)
    h3 = jnp.maximum(jnp.dot(h, wh1_r[...], preferred_element_type=jnp.float32)
                     + bh1_r[...], 0.0)
    out3 = jnp.dot(h3, wh2_r[...], preferred_element_type=jnp.float32) + bh2_r[...]
    prop_o[...] = jax.nn.sigmoid(out3[:, 0:1])
    k_o[...] = jnp.maximum(_softplus(out3[:, 1:2]), 0.01)
    l_o[...] = jnp.maximum(_softplus(out3[:, 2:3]), 0.01)
    wide_o[...] = wide


def _dense_call(deep4, wide4, g4, bt4, W1p, b1, W2, b2,
                Wh1, bh1, Wh2, bh2, Wwp, bb):
    nr = deep4.shape[0]
    b = nr * 8
    bbr = bb // 8
    grid = (b // bb,)

    def full(shape):
        return pl.BlockSpec(shape, lambda i: tuple(0 for _ in shape))

    return pl.pallas_call(
        _dense_body,
        grid=grid,
        in_specs=[
            pl.BlockSpec((bbr, LANE_TILES, 8, 128), lambda i: (i, 0, 0, 0)),
            pl.BlockSpec((bbr, LANE_TILES, 8, 128), lambda i: (i, 0, 0, 0)),
            full((LANE_TILES, 128)), full((LANE_TILES, 128)),
            full((LANE_TILES, 128, 512)), full((512,)),
            full((512, 256)), full((256,)),
            full((256, 384)), full((384,)),
            full((384, 3)), full((3,)),
            full((LANE_TILES, 128)),
        ],
        out_specs=[
            pl.BlockSpec((bb, 1), lambda i: (i, 0)),
            pl.BlockSpec((bb, 1), lambda i: (i, 0)),
            pl.BlockSpec((bb, 1), lambda i: (i, 0)),
            pl.BlockSpec((bb, 1), lambda i: (i, 0)),
        ],
        out_shape=[jax.ShapeDtypeStruct((b, 1), jnp.float32)] * 4,
    )(deep4, wide4, g4, bt4, W1p, b1, W2, b2, Wh1, bh1, Wh2, bh2, Wwp)


def kernel(x, deep_table, wide_table, ln_gamma, ln_beta, W1, b1, W2, b2,
           Wp1, bp1, Wp2, bp2, Wk1, bk1, Wk2, bk2, Wl1, bl1, Wl2, bl2, Wwide):
    b, f = x.shape
    inn = f * D
    padl = LANE_TILES * 128 - inn
    xpad = jnp.concatenate([x, x[:, : FP - f]], axis=1)
    pidx = _swizzle(xpad.reshape(b // 8, 8, LANE_TILES, 8)
                    .transpose(0, 2, 1, 3).reshape(-1))
    n_slots = b * FP

    eye8 = jnp.eye(128, dtype=jnp.float32).reshape(8, D, 128)
    deep_img = _relayout(deep_table.T, eye8)
    wide_img = _relayout(wide_table.T, eye8)
    v_pad = deep_img.shape[0] * 128 // D
    deep_lin = deep_img.reshape(v_pad, D)
    wide_lin = wide_img.reshape(v_pad, D)

    gather = _make_gather(n_slots, 1024)
    deep2, wide2 = gather(pidx, deep_lin, wide_lin)
    deep4 = deep2.reshape(b // 8, LANE_TILES, 8, 128)
    wide4 = wide2.reshape(b // 8, LANE_TILES, 8, 128)

    W1p = jnp.pad(W1, ((0, padl), (0, 0))).reshape(LANE_TILES, 128, 512)
    Wwp = jnp.pad(Wwide[:, 0], (0, padl)).reshape(LANE_TILES, 128)
    g4 = jnp.pad(ln_gamma, (0, padl)).reshape(LANE_TILES, 128)
    bt4 = jnp.pad(ln_beta, (0, padl)).reshape(LANE_TILES, 128)

    Wh1 = jnp.concatenate([Wp1, Wk1, Wl1], axis=1)
    bh1 = jnp.concatenate([bp1, bk1, bl1], axis=0)
    z = jnp.zeros((128, 1), jnp.float32)
    Wh2 = jnp.concatenate([
        jnp.concatenate([Wp2, z, z], axis=1),
        jnp.concatenate([z, Wk2, z], axis=1),
        jnp.concatenate([z, z, Wl2], axis=1),
    ], axis=0)
    bh2 = jnp.concatenate([bp2, bk2, bl2], axis=0)

    prop, k_p, l_p, wide = _dense_call(
        deep4, wide4, g4, bt4, W1p, b1, W2, b2,
        Wh1, bh1, Wh2, bh2, Wwp, 512)
    return (prop, k_p, l_p, wide)


# speedup vs baseline: 2.4035x; 1.1872x over previous
import functools

import jax
import jax.numpy as jnp
import numpy as np
from jax import lax
from jax.experimental import pallas as pl
from jax.experimental.pallas import tpu as pltpu
from jax.experimental.pallas import tpu_sc as plsc

D = 16
NC = 2
NS = 16
NW = NC * NS
LANE_TILES = 4
FP = LANE_TILES * 8


def _make_gather(n_slots, c):
    per_w = n_slots // NW
    n_chunks = per_w // c
    mesh = plsc.VectorSubcoreMesh(core_axis_name="c", subcore_axis_name="s")

    @functools.partial(
        pl.kernel,
        mesh=mesh,
        compiler_params=pltpu.CompilerParams(use_tc_tiling_on_sc=False),
        out_type=[
            jax.ShapeDtypeStruct((n_slots, D), jnp.float32),
            jax.ShapeDtypeStruct((n_slots, D), jnp.float32),
        ],
        scratch_types=[
            pltpu.VMEM((c,), jnp.int32),
            pltpu.VMEM((c,), jnp.int32),
            pltpu.VMEM((c, D), jnp.float32),
            pltpu.VMEM((c, D), jnp.float32),
            pltpu.VMEM((c, D), jnp.float32),
            pltpu.VMEM((c, D), jnp.float32),
            pltpu.SemaphoreType.DMA,
            pltpu.SemaphoreType.DMA,
            pltpu.SemaphoreType.DMA,
            pltpu.SemaphoreType.DMA,
        ],
    )
    def gather_k(idx_hbm, deep_hbm, wide_hbm, deep_out, wide_out,
                 i0, i1, d0, d1, w0, w1, sd0, sd1, sw0, sw1):
        wid = lax.axis_index("s") * NC + lax.axis_index("c")
        base = wid * per_w
        r_per_chunk = c // 1024
        ibufs = (i0, i1)
        dbufs = (d0, d1)
        wbufs = (w0, w1)
        sds = (sd0, sd1)
        sws = (sw0, sw1)
        handles = [None, None]

        def fire(g):
            b = g & 1
            off = base + g * c
            pltpu.sync_copy(idx_hbm.at[pl.ds(off, c)], ibufs[b])
            hd = pltpu.async_copy(deep_hbm.at[ibufs[b]], dbufs[b], sds[b])
            hw = pltpu.async_copy(wide_hbm.at[ibufs[b]], wbufs[b], sws[b])
            handles[b] = (hd, hw)

        fire(0)
        for g in range(n_chunks):
            if g + 1 < n_chunks:
                fire(g + 1)
            hd, hw = handles[g & 1]
            hd.wait()
            hw.wait()
            off = base + g * c
            pltpu.sync_copy(dbufs[g & 1], deep_out.at[pl.ds(off, c)])
            pltpu.sync_copy(wbufs[g & 1], wide_out.at[pl.ds(off, c)])

    return gather_k


RL_LANES = 4096


def _relayout_body(t_r, e_r, o_r):
    x = t_r[...]
    y = jnp.concatenate([x[:, u * 512:(u + 1) * 512] for u in range(8)],
                        axis=0)
    o_r[...] = lax.dot_general(
        y, e_r[...], (((0,), (0,)), ((), ())),
        preferred_element_type=jnp.float32)


def _relayout(table_t, eye8):
    d, v = table_t.shape
    grid = ((v + RL_LANES - 1) // RL_LANES,)
    rows = RL_LANES * d // 128
    return pl.pallas_call(
        _relayout_body,
        grid=grid,
        in_specs=[
            pl.BlockSpec((d, RL_LANES), lambda g: (0, g)),
            pl.BlockSpec((128, 128), lambda g: (0, 0)),
        ],
        out_specs=pl.BlockSpec((rows, 128), lambda g: (g, 0)),
        out_shape=jax.ShapeDtypeStruct((grid[0] * rows, 128), jnp.float32),
    )(table_t, eye8)


def _swizzle(r):
    rl = r & (RL_LANES - 1)
    return (r & ~(RL_LANES - 1)) | ((rl & 511) << 3) | (rl >> 9)


def _softplus(x):
    return jnp.maximum(x, 0.0) + jnp.log1p(jnp.exp(-jnp.abs(x)))


def _dense_body(de, we, g_r, bt_r, w1_r, b1_r, w2_r, b2_r, wh1_r, bh1_r,
                wh2_r, bh2_r, ww_r, prop_o, k_o, l_o, wide_o):
    bbr = de.shape[0]
    bb = bbr * 8
    inn = 416.0
    ej = [de[:, j].reshape(bb, 128) for j in range(LANE_TILES)]
    wj = [we[:, j].reshape(bb, 128) for j in range(LANE_TILES)]
    lane = lax.broadcasted_iota(jnp.int32, (bb, 128), 1)
    m3 = (lane < 32).astype(jnp.float32)

    s = (jnp.sum(ej[0], axis=-1, keepdims=True)
         + jnp.sum(ej[1], axis=-1, keepdims=True)
         + jnp.sum(ej[2], axis=-1, keepdims=True)
         + jnp.sum(ej[3] * m3, axis=-1, keepdims=True))
    mu = s / inn
    v = (jnp.sum((ej[0] - mu) ** 2, axis=-1, keepdims=True)
         + jnp.sum((ej[1] - mu) ** 2, axis=-1, keepdims=True)
         + jnp.sum((ej[2] - mu) ** 2, axis=-1, keepdims=True)
         + jnp.sum(((ej[3] - mu) ** 2) * m3, axis=-1, keepdims=True))
    var = v / inn
    rstd = 1.0 / jnp.sqrt(var + 1e-5)

    h = b1_r[...]
    wide = jnp.zeros((bb, 1), jnp.float32)
    for j in range(LANE_TILES):
        hj = (ej[j] - mu) * rstd * g_r[j] + bt_r[j]
        h = h + jnp.dot(hj, w1_r[j], preferred_element_type=jnp.float32)
        wide = wide + jnp.sum(wj[j] * ww_r[j], axis=-1, keepdims=True)
    h = jnp.maximum(h, 0.0)
    h = jnp.maximum(jnp.dot(h, w2_r[...], preferred_element_type=jnp.float32)
                    + b2_r[...], 0.0)
    h3 = jnp.maximum(jnp.dot(h, wh1_r[...], preferred_element_type=jnp.float32)
                     + bh1_r[...], 0.0)
    out3 = jnp.dot(h3, wh2_r[...], preferred_element_type=jnp.float32) + bh2_r[...]
    prop_o[...] = jax.nn.sigmoid(out3[:, 0:1])
    k_o[...] = jnp.maximum(_softplus(out3[:, 1:2]), 0.01)
    l_o[...] = jnp.maximum(_softplus(out3[:, 2:3]), 0.01)
    wide_o[...] = wide


def _dense_call(deep4, wide4, g4, bt4, W1p, b1, W2, b2,
                Wh1, bh1, Wh2, bh2, Wwp, bb):
    nr = deep4.shape[0]
    b = nr * 8
    bbr = bb // 8
    grid = (b // bb,)

    def full(shape):
        return pl.BlockSpec(shape, lambda i: tuple(0 for _ in shape))

    return pl.pallas_call(
        _dense_body,
        grid=grid,
        in_specs=[
            pl.BlockSpec((bbr, LANE_TILES, 8, 128), lambda i: (i, 0, 0, 0)),
            pl.BlockSpec((bbr, LANE_TILES, 8, 128), lambda i: (i, 0, 0, 0)),
            full((LANE_TILES, 128)), full((LANE_TILES, 128)),
            full((LANE_TILES, 128, 512)), full((512,)),
            full((512, 256)), full((256,)),
            full((256, 384)), full((384,)),
            full((384, 3)), full((3,)),
            full((LANE_TILES, 128)),
        ],
        out_specs=[
            pl.BlockSpec((bb, 1), lambda i: (i, 0)),
            pl.BlockSpec((bb, 1), lambda i: (i, 0)),
            pl.BlockSpec((bb, 1), lambda i: (i, 0)),
            pl.BlockSpec((bb, 1), lambda i: (i, 0)),
        ],
        out_shape=[jax.ShapeDtypeStruct((b, 1), jnp.float32)] * 4,
    )(deep4, wide4, g4, bt4, W1p, b1, W2, b2, Wh1, bh1, Wh2, bh2, Wwp)


def kernel(x, deep_table, wide_table, ln_gamma, ln_beta, W1, b1, W2, b2,
           Wp1, bp1, Wp2, bp2, Wk1, bk1, Wk2, bk2, Wl1, bl1, Wl2, bl2, Wwide):
    b, f = x.shape
    inn = f * D
    padl = LANE_TILES * 128 - inn
    xpad = jnp.concatenate([x, x[:, : FP - f]], axis=1)
    pidx = _swizzle(xpad.reshape(b // 8, 8, LANE_TILES, 8)
                    .transpose(0, 2, 1, 3).reshape(-1))
    n_slots = b * FP

    eye = jnp.eye(128, dtype=jnp.float32)
    deep_img = _relayout(deep_table.T, eye)
    wide_img = _relayout(wide_table.T, eye)
    v_pad = deep_img.shape[0] * 128 // D
    deep_lin = deep_img.reshape(v_pad, D)
    wide_lin = wide_img.reshape(v_pad, D)

    gather = _make_gather(n_slots, 1024)
    deep2, wide2 = gather(pidx, deep_lin, wide_lin)
    deep4 = deep2.reshape(b // 8, LANE_TILES, 8, 128)
    wide4 = wide2.reshape(b // 8, LANE_TILES, 8, 128)

    W1p = jnp.pad(W1, ((0, padl), (0, 0))).reshape(LANE_TILES, 128, 512)
    Wwp = jnp.pad(Wwide[:, 0], (0, padl)).reshape(LANE_TILES, 128)
    g4 = jnp.pad(ln_gamma, (0, padl)).reshape(LANE_TILES, 128)
    bt4 = jnp.pad(ln_beta, (0, padl)).reshape(LANE_TILES, 128)

    Wh1 = jnp.concatenate([Wp1, Wk1, Wl1], axis=1)
    bh1 = jnp.concatenate([bp1, bk1, bl1], axis=0)
    z = jnp.zeros((128, 1), jnp.float32)
    Wh2 = jnp.concatenate([
        jnp.concatenate([Wp2, z, z], axis=1),
        jnp.concatenate([z, Wk2, z], axis=1),
        jnp.concatenate([z, z, Wl2], axis=1),
    ], axis=0)
    bh2 = jnp.concatenate([bp2, bk2, bl2], axis=0)

    prop, k_p, l_p, wide = _dense_call(
        deep4, wide4, g4, bt4, W1p, b1, W2, b2,
        Wh1, bh1, Wh2, bh2, Wwp, 512)
    return (prop, k_p, l_p, wide)
